# baseline (speedup 1.0000x reference)
import jax
import jax.numpy as jnp
from jax.experimental import pallas as pl
from jax.experimental.pallas import tpu as pltpu
from jax.experimental.pallas import tpu_sc as plsc


_TN = 512
_K = 8192
_D = 32
_GW = 256

_S = float(10 ** 7)


def _argmin_kernel(x_ref, et_ref, i_ref, eq_ref, esq_ref, iota_ref):
    @pl.when(pl.program_id(0) == 0)
    def _init():
        eq = jnp.round(et_ref[...] * _S) / _S
        eq_ref[...] = eq
        esq = jnp.sum(eq * eq, axis=0, keepdims=True)
        esq_ref[...] = jnp.round(esq * _S) / _S
        iota_ref[...] = jax.lax.broadcasted_iota(
            jnp.int32, (_TN, _K), 1).astype(jnp.float32)

    x = x_ref[...]
    x_sq = jnp.sum(x * x, axis=1, keepdims=True)
    x_sq = jnp.round(x_sq * _S) / _S
    cross = jnp.dot(x, eq_ref[...], preferred_element_type=jnp.float32)
    c1 = jnp.round(cross * _S) / _S
    c2 = jnp.round(c1 * (2.0 * _S)) / _S
    dist = jnp.round((x_sq + esq_ref[...]) * _S) / _S
    dist = jnp.round((dist - c2) * _S) / _S
    m = jnp.min(dist, axis=1, keepdims=True)
    indf = jnp.min(jnp.where(dist == m, iota_ref[...], float(_K)),
                   axis=1)
    i_ref[...] = indf.astype(jnp.int32).reshape(1, 1, _TN)


def _codebook_indices(xf, et):
    n = xf.shape[0]
    grid = n // _TN
    ind = pl.pallas_call(
        _argmin_kernel,
        grid=(grid,),
        in_specs=[
            pl.BlockSpec((_TN, _D), lambda i: (i, 0)),
            pl.BlockSpec((_D, _K), lambda i: (0, 0)),
        ],
        out_specs=pl.BlockSpec((1, 1, _TN), lambda i: (i, 0, 0)),
        out_shape=jax.ShapeDtypeStruct((grid, 1, _TN), jnp.int32),
        scratch_shapes=[
            pltpu.VMEM((_D, _K), jnp.float32),
            pltpu.VMEM((1, _K), jnp.float32),
            pltpu.VMEM((_TN, _K), jnp.float32),
        ],
    )(xf, et)
    return ind.reshape(1, n)


def _gather_rows(embed_pad, ind_flat):
    n = ind_flat.shape[1]
    w = embed_pad.shape[1]

    @pl.kernel(
        out_type=jax.ShapeDtypeStruct((n, w), jnp.float32),
        mesh=plsc.VectorSubcoreMesh(core_axis_name="core",
                                    subcore_axis_name="subcore"),
    )
    def _k(e_hbm, i_hbm, o_hbm):
        def body(i_vmem, o_vmem):
            pltpu.sync_copy(e_hbm.at[i_vmem.at[0]], o_vmem)

        pltpu.emit_pipeline(
            body,
            grid=(n // _GW,),
            in_specs=[pl.BlockSpec((1, _GW), index_map=lambda i: (0, i))],
            out_specs=[pl.BlockSpec((_GW, w), index_map=lambda i: (i, 0))],
            core_axis_name=("core", "subcore"),
            dimension_semantics=(pltpu.PARALLEL,),
        )(i_hbm, o_hbm)

    return _k(embed_pad, ind_flat)


def kernel(x, embed):
    shape = x.shape
    n = shape[0] * shape[1]
    xf = x.reshape(n, shape[-1])
    et = embed.T
    embed_pad = jnp.pad(embed, ((0, 0), (0, 128 - _D)))
    half = n // 2
    ind1 = _codebook_indices(xf[:half], et)
    quant1 = _gather_rows(embed_pad, ind1)[:, :_D]
    ind2 = _codebook_indices(xf[half:], et)
    quant2 = _gather_rows(embed_pad, ind2)[:, :_D]
    quant = jnp.concatenate([quant1, quant2], axis=0)
    ind = jnp.concatenate([ind1, ind2], axis=1)
    return quant.reshape(shape), ind.reshape(shape[:-1])


# speedup vs baseline: 1.0787x; 1.0787x over previous
import jax
import jax.numpy as jnp
from jax.experimental import pallas as pl
from jax.experimental.pallas import tpu as pltpu
from jax.experimental.pallas import tpu_sc as plsc


_TN = 512
_K = 8192
_D = 32
_GW = 256

_S = float(10 ** 7)


def _argmin_kernel(x_ref, et_ref, i_ref, eq_ref, esq_ref):
    @pl.when(pl.program_id(0) == 0)
    def _init():
        eq = jnp.round(et_ref[...] * _S) / _S
        eq_ref[...] = eq
        esq = jnp.sum(eq * eq, axis=0, keepdims=True)
        esq_ref[...] = jnp.round(esq * _S) / _S

    x = x_ref[...]
    x_sq = jnp.sum(x * x, axis=1, keepdims=True)
    x_sq = jnp.round(x_sq * _S) / _S
    cross = jnp.dot(x, eq_ref[...], preferred_element_type=jnp.float32)
    c1 = jnp.round(cross * _S) / _S
    c2 = jnp.round(c1 * (2.0 * _S)) / _S
    dist = jnp.round((x_sq + esq_ref[...]) * _S) / _S
    dist = jnp.round((dist - c2) * _S) / _S
    ind = jnp.argmin(dist, axis=1).astype(jnp.int32)
    i_ref[...] = ind.reshape(1, 1, _TN)


def _codebook_indices(xf, et):
    n = xf.shape[0]
    grid = n // _TN
    ind = pl.pallas_call(
        _argmin_kernel,
        grid=(grid,),
        in_specs=[
            pl.BlockSpec((_TN, _D), lambda i: (i, 0)),
            pl.BlockSpec((_D, _K), lambda i: (0, 0)),
        ],
        out_specs=pl.BlockSpec((1, 1, _TN), lambda i: (i, 0, 0)),
        out_shape=jax.ShapeDtypeStruct((grid, 1, _TN), jnp.int32),
        scratch_shapes=[
            pltpu.VMEM((_D, _K), jnp.float32),
            pltpu.VMEM((1, _K), jnp.float32),
        ],
    )(xf, et)
    return ind.reshape(1, n)


def _gather_rows(embed_pad, ind_flat):
    n = ind_flat.shape[1]
    w = embed_pad.shape[1]

    @pl.kernel(
        out_type=jax.ShapeDtypeStruct((n, w), jnp.float32),
        mesh=plsc.VectorSubcoreMesh(core_axis_name="core",
                                    subcore_axis_name="subcore"),
    )
    def _k(e_hbm, i_hbm, o_hbm):
        def body(i_vmem, o_vmem):
            pltpu.sync_copy(e_hbm.at[i_vmem.at[0]], o_vmem)

        pltpu.emit_pipeline(
            body,
            grid=(n // _GW,),
            in_specs=[pl.BlockSpec((1, _GW), index_map=lambda i: (0, i))],
            out_specs=[pl.BlockSpec((_GW, w), index_map=lambda i: (i, 0))],
            core_axis_name=("core", "subcore"),
            dimension_semantics=(pltpu.PARALLEL,),
        )(i_hbm, o_hbm)

    return _k(embed_pad, ind_flat)


def kernel(x, embed):
    shape = x.shape
    n = shape[0] * shape[1]
    xf = x.reshape(n, shape[-1])
    et = embed.T
    embed_pad = jnp.pad(embed, ((0, 0), (0, 128 - _D)))
    ind = _codebook_indices(xf, et)
    quant = _gather_rows(embed_pad, ind)[:, :_D]
    return quant.reshape(shape), ind.reshape(shape[:-1])
